# baseline (speedup 1.0000x reference)
import functools

import jax
import jax.numpy as jnp
from jax import lax
from jax.experimental import pallas as pl
from jax.experimental.pallas import tpu as pltpu
from jax.experimental.pallas import tpu_sc as plsc

N_NODES = 100000
N_EDGES = 1600000
NP = 100352
ROWS_PER_TILE = NP // 16
E_PAD = 1638400
EDGES_PER_TILE = E_PAD // 16
CHUNK = 512
N_CHUNKS = EDGES_PER_TILE // CHUNK
N_PAIRS = N_CHUNKS // 2
GROUPS = CHUNK // 16
IDXW = 128
N_SLICES = CHUNK // IDXW
PAD_ROWS = 2048
TAB_ROWS = NP + PAD_ROWS
DT = 0.01
ODE_H = 0.005


def _sc_segment_sums(e_flat, sidx2d, ridx2d, w1b, b1b, zrow, zcnt):
    mesh = plsc.VectorSubcoreMesh(core_axis_name="c", subcore_axis_name="s")
    f32 = jnp.float32

    @functools.partial(
        pl.kernel,
        out_type=(
            jax.ShapeDtypeStruct((NP, 16), f32),
            jax.ShapeDtypeStruct((NP,), f32),
            jax.ShapeDtypeStruct((NP, 16), f32),
            jax.ShapeDtypeStruct((NP,), f32),
        ),
        mesh=mesh,
        compiler_params=pltpu.CompilerParams(use_tc_tiling_on_sc=False),
        scratch_types=(
            pltpu.VMEM_SHARED((TAB_ROWS, 16), f32),
            pltpu.VMEM_SHARED((TAB_ROWS,), f32),
            pltpu.VMEM((2, CHUNK, 16), f32),
            pltpu.VMEM((2, CHUNK), f32),
            pltpu.VMEM((2, N_SLICES, IDXW), jnp.int32),
            pltpu.VMEM((IDXW,), f32),
            pltpu.VMEM((16,), f32),
            pltpu.VMEM((16,), f32),
            pltpu.SemaphoreType.DMA,
            pltpu.SemaphoreType.DMA,
            pltpu.SemaphoreType.DMA,
            pltpu.SemaphoreType.DMA,
        ),
    )
    def sck(ev_hbm, sidx_hbm, ridx_hbm, w1_hbm, b1_hbm, zrow_hbm, zcnt_hbm,
            as_out, cs_out, ar_out, cr_out,
            table, cnt, rows_v, ev_v, idx_v, ones_v, w1_v, b1_v,
            sem_in0, sem_in1, sem_sc0, sem_sc1):
        c = lax.axis_index("c")
        s = lax.axis_index("s")

        pltpu.sync_copy(w1_hbm, w1_v)
        pltpu.sync_copy(b1_hbm, b1_v)
        for j in range(IDXW // 16):
            ones_v[pl.ds(j * 16, 16)] = jnp.ones((16,), f32)

        row0 = s * ROWS_PER_TILE
        pltpu.sync_copy(zrow_hbm, table.at[pl.ds(row0, ROWS_PER_TILE)])
        pltpu.sync_copy(zcnt_hbm, cnt.at[pl.ds(row0, ROWS_PER_TILE)])
        plsc.subcore_barrier()

        w1v = w1_v[...]
        b1v = b1_v[...]

        dn = lax.GatherDimensionNumbers(
            offset_dims=(), collapsed_slice_dims=(0,),
            start_index_map=(0,))
        lane_idx = [jnp.full((16, 1), t, jnp.int32) for t in range(16)]
        sem_in = (sem_in0, sem_in1)
        sem_sc = (sem_sc0, sem_sc1)
        ebase = s * EDGES_PER_TILE
        ibase = s * (EDGES_PER_TILE // IDXW)

        def do_edges(idx_hbm):
            def fire_in(b, ck):
                pltpu.async_copy(
                    ev_hbm.at[pl.ds(ebase + ck * CHUNK, CHUNK)],
                    ev_v.at[b], sem_in[b])
                pltpu.async_copy(
                    idx_hbm.at[pl.ds(ibase + ck * N_SLICES, N_SLICES)],
                    idx_v.at[b], sem_in[b])

            def wait_in(b):
                pltpu.make_async_copy(
                    ev_hbm.at[pl.ds(0, CHUNK)], ev_v.at[b],
                    sem_in[b]).wait()
                pltpu.make_async_copy(
                    idx_hbm.at[pl.ds(0, N_SLICES)], idx_v.at[b],
                    sem_in[b]).wait()

            def compute(b):
                def e_body(g, carry2):
                    ev16 = ev_v[b, pl.ds(g * 16, 16)]
                    for t in range(16):
                        evb = lax.gather(
                            ev16, lane_idx[t], dn, (1,),
                            mode=lax.GatherScatterMode.PROMISE_IN_BOUNDS)
                        rows_v[b, g * 16 + t] = jnp.maximum(
                            evb * w1v + b1v, 0.0)
                    return carry2
                lax.fori_loop(0, GROUPS, e_body, 0, unroll=2)

            def fire_scatters(b):
                descs = []
                for j in range(N_SLICES):
                    descs.append(pltpu.async_copy(
                        rows_v.at[b].at[pl.ds(j * IDXW, IDXW)],
                        table.at[idx_v.at[b].at[j]], sem_sc[b], add=True))
                    descs.append(pltpu.async_copy(
                        ones_v, cnt.at[idx_v.at[b].at[j]], sem_sc[b],
                        add=True))
                return descs

            fire_in(0, 0)
            fire_in(1, 1)

            def pair_body(pi, carry):
                wait_in(0)
                compute(0)
                d0 = fire_scatters(0)
                wait_in(1)
                compute(1)
                d1 = fire_scatters(1)
                for d in d0:
                    d.wait()

                @pl.when(pi < N_PAIRS - 1)
                def _():
                    fire_in(0, 2 * pi + 2)
                for d in d1:
                    d.wait()

                @pl.when(pi < N_PAIRS - 1)
                def _():
                    fire_in(1, 2 * pi + 3)
                return carry
            lax.fori_loop(0, N_PAIRS, pair_body, 0, unroll=False)

        @pl.when(c == 0)
        def _():
            do_edges(sidx_hbm)

        @pl.when(c == 1)
        def _():
            do_edges(ridx_hbm)

        plsc.subcore_barrier()

        @pl.when(c == 0)
        def _():
            pltpu.sync_copy(table.at[pl.ds(row0, ROWS_PER_TILE)],
                            as_out.at[pl.ds(row0, ROWS_PER_TILE)])
            pltpu.sync_copy(cnt.at[pl.ds(row0, ROWS_PER_TILE)],
                            cs_out.at[pl.ds(row0, ROWS_PER_TILE)])

        @pl.when(c == 1)
        def _():
            pltpu.sync_copy(table.at[pl.ds(row0, ROWS_PER_TILE)],
                            ar_out.at[pl.ds(row0, ROWS_PER_TILE)])
            pltpu.sync_copy(cnt.at[pl.ds(row0, ROWS_PER_TILE)],
                            cr_out.at[pl.ds(row0, ROWS_PER_TILE)])

    return sck(e_flat, sidx2d, ridx2d, w1b, b1b, zrow, zcnt)


def _mmT(wT, x):
    return lax.dot_general(wT, x, (((1,), (0,)), ((), ())),
                           precision=lax.Precision.DEFAULT,
                           preferred_element_type=jnp.float32)


def _mmA(wT, a):
    return lax.dot_general(wT, a, (((1,), (1,)), ((), ())),
                           precision=lax.Precision.DEFAULT,
                           preferred_element_type=jnp.float32)


def _tc_body(nodesT_ref, as_ref, cs_ref, ar_ref, cr_ref, g_ref,
             we1_ref, be1_ref, we2_ref, be2_ref,
             w2e_ref, b2e_ref,
             wd1_ref, bd1_ref, wd2_ref, bd2_ref, wd3_ref, bd3_ref,
             wp_ref, bp_ref,
             wc1_ref, bc1_ref, wc2_ref, bc2_ref, wc3_ref, bc3_ref,
             out_ref):
    B = nodesT_ref.shape[1]
    nodesT = nodesT_ref[...]

    h = jnp.maximum(_mmT(we1_ref[...], nodesT[:6]) + be1_ref[...], 0.0)
    hT = _mmT(we2_ref[...], h) + be2_ref[...]

    b2e = b2e_ref[...]
    sentT = _mmA(w2e_ref[...], as_ref[...]) + b2e * cs_ref[0]
    recvT = _mmA(w2e_ref[...], ar_ref[...]) + b2e * cr_ref[0]

    gB = jnp.broadcast_to(g_ref[...], (2, B))
    x = jnp.concatenate([hT, sentT, recvT, gB], axis=0)

    wd1 = wd1_ref[...]; bd1 = bd1_ref[...]
    wd2 = wd2_ref[...]; bd2 = bd2_ref[...]
    wd3 = wd3_ref[...]; bd3 = bd3_ref[...]

    def f(y):
        z1 = jnp.maximum(_mmT(wd1, y) + bd1, 0.0)
        z2 = jnp.maximum(_mmT(wd2, z1) + bd2, 0.0)
        return _mmT(wd3, z2) + bd3

    for _ in range(2):
        k1 = f(x)
        k2 = f(x + (0.5 * ODE_H) * k1)
        k3 = f(x + (0.5 * ODE_H) * k2)
        k4 = f(x + ODE_H * k3)
        x = x + (ODE_H / 6.0) * (k1 + 2.0 * k2 + 2.0 * k3 + k4)

    z = _mmT(wp_ref[...], x) + bp_ref[...] + hT

    d1 = jnp.maximum(_mmT(wc1_ref[...], z) + bc1_ref[...], 0.0)
    d2 = jnp.maximum(_mmT(wc2_ref[...], d1) + bc2_ref[...], 0.0)
    acc = _mmT(wc3_ref[...], d2) + bc3_ref[...]

    next_vel = nodesT[5:6] + acc * DT
    next_pos = nodesT[0:1] + next_vel * DT
    out_ref[...] = jnp.concatenate(
        [next_pos, nodesT[2:6], next_vel, acc], axis=0)


TBLK = 14336


def _tc_pipeline(nodesT, a_s, c_s, a_r, c_r, g2, wts):
    blk = TBLK
    grid = (NP // blk,)
    full = lambda arr: pl.BlockSpec(arr.shape, lambda i: (0,) * arr.ndim)
    in_specs = [
        pl.BlockSpec((7, blk), lambda i: (0, i)),
        pl.BlockSpec((blk, 16), lambda i: (i, 0)),
        pl.BlockSpec((1, 1, blk), lambda i: (i, 0, 0)),
        pl.BlockSpec((blk, 16), lambda i: (i, 0)),
        pl.BlockSpec((1, 1, blk), lambda i: (i, 0, 0)),
        full(g2),
    ] + [full(w) for w in wts]
    return pl.pallas_call(
        _tc_body,
        grid=grid,
        in_specs=in_specs,
        out_specs=pl.BlockSpec((7, blk), lambda i: (0, i)),
        out_shape=jax.ShapeDtypeStruct((7, NP), jnp.float32),
    )(nodesT, a_s, c_s, a_r, c_r, g2, *wts)


def kernel(nodes, edges, globals_, params, senders, receivers):
    f32 = jnp.float32

    npad = E_PAD - N_EDGES
    e_flat = jnp.concatenate(
        [edges.reshape(-1).astype(f32), jnp.zeros((npad,), f32)])
    pad_idx = NP + (jnp.arange(npad, dtype=jnp.int32) % PAD_ROWS)
    sidx2d = jnp.concatenate(
        [senders.astype(jnp.int32), pad_idx]).reshape(E_PAD // IDXW, IDXW)
    ridx2d = jnp.concatenate(
        [receivers.astype(jnp.int32), pad_idx]).reshape(E_PAD // IDXW, IDXW)

    (we0, be0), (we1, be1) = params["enc_edge"]
    w1 = we0.reshape(16).astype(f32)
    b1 = be0.reshape(16).astype(f32)
    zrow = jnp.zeros((ROWS_PER_TILE, 16), f32)
    zcnt = jnp.zeros((ROWS_PER_TILE,), f32)

    a_s, c_s, a_r, c_r = _sc_segment_sums(
        e_flat, sidx2d, ridx2d, w1, b1, zrow, zcnt)

    nodes7 = jnp.zeros((NP, 7), f32).at[:N_NODES, :6].set(nodes)
    nodesT = nodes7.T
    cs3 = c_s.reshape(NP // TBLK, 1, TBLK)
    cr3 = c_r.reshape(NP // TBLK, 1, TBLK)
    g2 = globals_.reshape(2, 1).astype(f32)

    (wn0, bn0), (wn1, bn1) = params["enc_node"]
    (wd0, bd0), (wd1, bd1), (wd2, bd2) = params["deriv"]
    ((wp0, bp0),) = params["proc_out"]
    (wc0, bc0), (wc1, bc1), (wc2, bc2) = params["dec_node"]
    col = lambda b: b.reshape(-1, 1).astype(f32)
    wts = [
        wn0.T.astype(f32), col(bn0), wn1.T.astype(f32), col(bn1),
        we1.T.astype(f32), col(be1),
        wd0.T.astype(f32), col(bd0), wd1.T.astype(f32), col(bd1),
        wd2.T.astype(f32), col(bd2),
        wp0.T.astype(f32), col(bp0),
        wc0.T.astype(f32), col(bc0), wc1.T.astype(f32), col(bc1),
        wc2.T.astype(f32), col(bc2),
    ]

    outT = _tc_pipeline(nodesT, a_s, cs3, a_r, cr3, g2, wts)

    next_nodes = outT[:, :N_NODES].T
    next_pos = outT[0, :N_NODES]
    next_edges = jnp.diff(next_pos).reshape(-1, 1)
    new_globals = jnp.concatenate(
        [jnp.reshape(globals_[0] + 1.0, (1,)), globals_[1:]])
    return next_nodes, next_edges, new_globals


# speedup vs baseline: 1.0140x; 1.0140x over previous
import functools

import jax
import jax.numpy as jnp
from jax import lax
from jax.experimental import pallas as pl
from jax.experimental.pallas import tpu as pltpu
from jax.experimental.pallas import tpu_sc as plsc

N_NODES = 100000
N_EDGES = 1600000
NP = 100352
ROWS_PER_TILE = NP // 16
E_PAD = 1638400
EDGES_PER_TILE = E_PAD // 16
CHUNK = 512
N_CHUNKS = EDGES_PER_TILE // CHUNK
N_PAIRS = N_CHUNKS // 2
GROUPS = CHUNK // 16
IDXW = 128
N_SLICES = CHUNK // IDXW
PAD_ROWS = 2048
TAB_ROWS = NP + PAD_ROWS
DT = 0.01
ODE_H = 0.005


def _sc_segment_sums(e_flat, sidx2d, ridx2d, w1b, b1b, zrow, zcnt):
    mesh = plsc.VectorSubcoreMesh(core_axis_name="c", subcore_axis_name="s")
    f32 = jnp.float32

    @functools.partial(
        pl.kernel,
        out_type=(
            jax.ShapeDtypeStruct((NP, 16), f32),
            jax.ShapeDtypeStruct((NP,), f32),
            jax.ShapeDtypeStruct((NP, 16), f32),
            jax.ShapeDtypeStruct((NP,), f32),
        ),
        mesh=mesh,
        compiler_params=pltpu.CompilerParams(use_tc_tiling_on_sc=False),
        scratch_types=(
            pltpu.VMEM_SHARED((TAB_ROWS, 16), f32),
            pltpu.VMEM_SHARED((TAB_ROWS,), f32),
            pltpu.VMEM((2, CHUNK, 16), f32),
            pltpu.VMEM((2, CHUNK), f32),
            pltpu.VMEM((2, N_SLICES, IDXW), jnp.int32),
            pltpu.VMEM((IDXW,), f32),
            pltpu.VMEM((16,), f32),
            pltpu.VMEM((16,), f32),
            pltpu.SemaphoreType.DMA,
            pltpu.SemaphoreType.DMA,
            pltpu.SemaphoreType.DMA,
            pltpu.SemaphoreType.DMA,
        ),
    )
    def sck(ev_hbm, sidx_hbm, ridx_hbm, w1_hbm, b1_hbm, zrow_hbm, zcnt_hbm,
            as_out, cs_out, ar_out, cr_out,
            table, cnt, rows_v, ev_v, idx_v, ones_v, w1_v, b1_v,
            sem_in0, sem_in1, sem_sc0, sem_sc1):
        c = lax.axis_index("c")
        s = lax.axis_index("s")

        pltpu.sync_copy(w1_hbm, w1_v)
        pltpu.sync_copy(b1_hbm, b1_v)
        for j in range(IDXW // 16):
            ones_v[pl.ds(j * 16, 16)] = jnp.ones((16,), f32)

        row0 = s * ROWS_PER_TILE
        pltpu.sync_copy(zrow_hbm, table.at[pl.ds(row0, ROWS_PER_TILE)])
        pltpu.sync_copy(zcnt_hbm, cnt.at[pl.ds(row0, ROWS_PER_TILE)])
        plsc.subcore_barrier()

        w1v = w1_v[...]
        b1v = b1_v[...]

        dn = lax.GatherDimensionNumbers(
            offset_dims=(), collapsed_slice_dims=(0,),
            start_index_map=(0,))
        lane_idx = [jnp.full((16, 1), t, jnp.int32) for t in range(16)]
        sem_in = (sem_in0, sem_in1)
        sem_sc = (sem_sc0, sem_sc1)
        ebase = s * EDGES_PER_TILE
        ibase = s * (EDGES_PER_TILE // IDXW)

        def do_edges(idx_hbm):
            def fire_in(b, ck):
                pltpu.async_copy(
                    ev_hbm.at[pl.ds(ebase + ck * CHUNK, CHUNK)],
                    ev_v.at[b], sem_in[b])
                pltpu.async_copy(
                    idx_hbm.at[pl.ds(ibase + ck * N_SLICES, N_SLICES)],
                    idx_v.at[b], sem_in[b])

            def wait_in(b):
                pltpu.make_async_copy(
                    ev_hbm.at[pl.ds(0, CHUNK)], ev_v.at[b],
                    sem_in[b]).wait()
                pltpu.make_async_copy(
                    idx_hbm.at[pl.ds(0, N_SLICES)], idx_v.at[b],
                    sem_in[b]).wait()

            def compute(b):
                def e_body(g, carry2):
                    ev16 = ev_v[b, pl.ds(g * 16, 16)]
                    for t in range(16):
                        evb = lax.gather(
                            ev16, lane_idx[t], dn, (1,),
                            mode=lax.GatherScatterMode.PROMISE_IN_BOUNDS)
                        rows_v[b, g * 16 + t] = jnp.maximum(
                            evb * w1v + b1v, 0.0)
                    return carry2
                lax.fori_loop(0, GROUPS, e_body, 0, unroll=False)

            def fire_scatters(b):
                descs = []
                for j in range(N_SLICES):
                    descs.append(pltpu.async_copy(
                        rows_v.at[b].at[pl.ds(j * IDXW, IDXW)],
                        table.at[idx_v.at[b].at[j]], sem_sc[b], add=True))
                    descs.append(pltpu.async_copy(
                        ones_v, cnt.at[idx_v.at[b].at[j]], sem_sc[b],
                        add=True))
                return descs

            fire_in(0, 0)
            fire_in(1, 1)

            def pair_body(pi, carry):
                wait_in(0)
                compute(0)
                d0 = fire_scatters(0)
                wait_in(1)
                compute(1)
                d1 = fire_scatters(1)
                for d in d0:
                    d.wait()

                @pl.when(pi < N_PAIRS - 1)
                def _():
                    fire_in(0, 2 * pi + 2)
                for d in d1:
                    d.wait()

                @pl.when(pi < N_PAIRS - 1)
                def _():
                    fire_in(1, 2 * pi + 3)
                return carry
            lax.fori_loop(0, N_PAIRS, pair_body, 0, unroll=False)

        @pl.when(c == 0)
        def _():
            do_edges(sidx_hbm)

        @pl.when(c == 1)
        def _():
            do_edges(ridx_hbm)

        plsc.subcore_barrier()

        @pl.when(c == 0)
        def _():
            pltpu.sync_copy(table.at[pl.ds(row0, ROWS_PER_TILE)],
                            as_out.at[pl.ds(row0, ROWS_PER_TILE)])
            pltpu.sync_copy(cnt.at[pl.ds(row0, ROWS_PER_TILE)],
                            cs_out.at[pl.ds(row0, ROWS_PER_TILE)])

        @pl.when(c == 1)
        def _():
            pltpu.sync_copy(table.at[pl.ds(row0, ROWS_PER_TILE)],
                            ar_out.at[pl.ds(row0, ROWS_PER_TILE)])
            pltpu.sync_copy(cnt.at[pl.ds(row0, ROWS_PER_TILE)],
                            cr_out.at[pl.ds(row0, ROWS_PER_TILE)])

    return sck(e_flat, sidx2d, ridx2d, w1b, b1b, zrow, zcnt)


def _mmT(wT, x):
    return lax.dot_general(wT, x, (((1,), (0,)), ((), ())),
                           precision=lax.Precision.DEFAULT,
                           preferred_element_type=jnp.float32)


def _mmA(wT, a):
    return lax.dot_general(wT, a, (((1,), (1,)), ((), ())),
                           precision=lax.Precision.DEFAULT,
                           preferred_element_type=jnp.float32)


def _tc_body(nodesT_ref, as_ref, cs_ref, ar_ref, cr_ref, g_ref,
             we1_ref, be1_ref, we2_ref, be2_ref,
             w2e_ref, b2e_ref,
             wd1_ref, bd1_ref, wd2_ref, bd2_ref, wd3_ref, bd3_ref,
             wp_ref, bp_ref,
             wc1_ref, bc1_ref, wc2_ref, bc2_ref, wc3_ref, bc3_ref,
             out_ref):
    B = nodesT_ref.shape[1]
    nodesT = nodesT_ref[...]

    h = jnp.maximum(_mmT(we1_ref[...], nodesT[:6]) + be1_ref[...], 0.0)
    hT = _mmT(we2_ref[...], h) + be2_ref[...]

    b2e = b2e_ref[...]
    sentT = _mmA(w2e_ref[...], as_ref[...]) + b2e * cs_ref[0]
    recvT = _mmA(w2e_ref[...], ar_ref[...]) + b2e * cr_ref[0]

    gB = jnp.broadcast_to(g_ref[...], (2, B))
    x = jnp.concatenate([hT, sentT, recvT, gB], axis=0)

    wd1 = wd1_ref[...]; bd1 = bd1_ref[...]
    wd2 = wd2_ref[...]; bd2 = bd2_ref[...]
    wd3 = wd3_ref[...]; bd3 = bd3_ref[...]

    def f(y):
        z1 = jnp.maximum(_mmT(wd1, y) + bd1, 0.0)
        z2 = jnp.maximum(_mmT(wd2, z1) + bd2, 0.0)
        return _mmT(wd3, z2) + bd3

    for _ in range(2):
        k1 = f(x)
        k2 = f(x + (0.5 * ODE_H) * k1)
        k3 = f(x + (0.5 * ODE_H) * k2)
        k4 = f(x + ODE_H * k3)
        x = x + (ODE_H / 6.0) * (k1 + 2.0 * k2 + 2.0 * k3 + k4)

    z = _mmT(wp_ref[...], x) + bp_ref[...] + hT

    d1 = jnp.maximum(_mmT(wc1_ref[...], z) + bc1_ref[...], 0.0)
    d2 = jnp.maximum(_mmT(wc2_ref[...], d1) + bc2_ref[...], 0.0)
    acc = _mmT(wc3_ref[...], d2) + bc3_ref[...]

    next_vel = nodesT[5:6] + acc * DT
    next_pos = nodesT[0:1] + next_vel * DT
    out_ref[...] = jnp.concatenate(
        [next_pos, nodesT[2:6], next_vel, acc], axis=0)


TBLK = 14336


def _tc_pipeline(nodesT, a_s, c_s, a_r, c_r, g2, wts):
    blk = TBLK
    grid = (NP // blk,)
    full = lambda arr: pl.BlockSpec(arr.shape, lambda i: (0,) * arr.ndim)
    in_specs = [
        pl.BlockSpec((7, blk), lambda i: (0, i)),
        pl.BlockSpec((blk, 16), lambda i: (i, 0)),
        pl.BlockSpec((1, 1, blk), lambda i: (i, 0, 0)),
        pl.BlockSpec((blk, 16), lambda i: (i, 0)),
        pl.BlockSpec((1, 1, blk), lambda i: (i, 0, 0)),
        full(g2),
    ] + [full(w) for w in wts]
    return pl.pallas_call(
        _tc_body,
        grid=grid,
        in_specs=in_specs,
        out_specs=pl.BlockSpec((7, blk), lambda i: (0, i)),
        out_shape=jax.ShapeDtypeStruct((7, NP), jnp.float32),
    )(nodesT, a_s, c_s, a_r, c_r, g2, *wts)


def kernel(nodes, edges, globals_, params, senders, receivers):
    f32 = jnp.float32

    npad = E_PAD - N_EDGES
    e_flat = jnp.concatenate(
        [edges.reshape(-1).astype(f32), jnp.zeros((npad,), f32)])
    pad_idx = NP + (jnp.arange(npad, dtype=jnp.int32) % PAD_ROWS)
    sidx2d = jnp.concatenate(
        [senders.astype(jnp.int32), pad_idx]).reshape(E_PAD // IDXW, IDXW)
    ridx2d = jnp.concatenate(
        [receivers.astype(jnp.int32), pad_idx]).reshape(E_PAD // IDXW, IDXW)

    (we0, be0), (we1, be1) = params["enc_edge"]
    w1 = we0.reshape(16).astype(f32)
    b1 = be0.reshape(16).astype(f32)
    zrow = jnp.zeros((ROWS_PER_TILE, 16), f32)
    zcnt = jnp.zeros((ROWS_PER_TILE,), f32)

    a_s, c_s, a_r, c_r = _sc_segment_sums(
        e_flat, sidx2d, ridx2d, w1, b1, zrow, zcnt)

    nodes7 = jnp.zeros((NP, 7), f32).at[:N_NODES, :6].set(nodes)
    nodesT = nodes7.T
    cs3 = c_s.reshape(NP // TBLK, 1, TBLK)
    cr3 = c_r.reshape(NP // TBLK, 1, TBLK)
    g2 = globals_.reshape(2, 1).astype(f32)

    (wn0, bn0), (wn1, bn1) = params["enc_node"]
    (wd0, bd0), (wd1, bd1), (wd2, bd2) = params["deriv"]
    ((wp0, bp0),) = params["proc_out"]
    (wc0, bc0), (wc1, bc1), (wc2, bc2) = params["dec_node"]
    col = lambda b: b.reshape(-1, 1).astype(f32)
    wts = [
        wn0.T.astype(f32), col(bn0), wn1.T.astype(f32), col(bn1),
        we1.T.astype(f32), col(be1),
        wd0.T.astype(f32), col(bd0), wd1.T.astype(f32), col(bd1),
        wd2.T.astype(f32), col(bd2),
        wp0.T.astype(f32), col(bp0),
        wc0.T.astype(f32), col(bc0), wc1.T.astype(f32), col(bc1),
        wc2.T.astype(f32), col(bc2),
    ]

    outT = _tc_pipeline(nodesT, a_s, cs3, a_r, cr3, g2, wts)

    next_nodes = outT[:, :N_NODES].T
    next_pos = outT[0, :N_NODES]
    next_edges = jnp.diff(next_pos).reshape(-1, 1)
    new_globals = jnp.concatenate(
        [jnp.reshape(globals_[0] + 1.0, (1,)), globals_[1:]])
    return next_nodes, next_edges, new_globals
